# speedup vs baseline: 3.6877x; 3.6877x over previous
import functools

import jax
import jax.numpy as jnp
from jax import lax
from jax.experimental import pallas as pl
from jax.experimental.pallas import tpu as pltpu
from jax.experimental.pallas import tpu_sc as plsc

N_FIELDS = 26
VOCAB_P1 = 1001
EMB = 128
BATCH = 16384
DENSE_DIM = 13
OUT_DIM = N_FIELDS * EMB + DENSE_DIM

NC, NS = 2, 16
NW = NC * NS
W = BATCH // NW
CHUNK = 128
NCHUNK = W // CHUNK


def _zero_pad_row(tables):
    def body(t_ref, o_ref):
        row = lax.broadcasted_iota(jnp.int32, (1, VOCAB_P1, EMB), 1)
        o_ref[...] = jnp.where(row == 0, 0.0, t_ref[...])

    return pl.pallas_call(
        body,
        grid=(N_FIELDS,),
        in_specs=[pl.BlockSpec((1, VOCAB_P1, EMB), lambda i: (i, 0, 0))],
        out_specs=pl.BlockSpec((1, VOCAB_P1, EMB), lambda i: (i, 0, 0)),
        out_shape=jax.ShapeDtypeStruct((N_FIELDS, VOCAB_P1, EMB), jnp.float32),
    )(tables)


def _sc_embed(feats, dense, t):
    mesh = plsc.VectorSubcoreMesh(core_axis_name="c", subcore_axis_name="s")

    @functools.partial(
        pl.kernel,
        out_type=jax.ShapeDtypeStruct((BATCH, OUT_DIM), jnp.float32),
        mesh=mesh,
        scratch_types=[
            pltpu.VMEM((W,), jnp.int32),
            pltpu.VMEM((CHUNK, EMB), jnp.float32),
            pltpu.VMEM((W, DENSE_DIM), jnp.float32),
            pltpu.SemaphoreType.DMA,
        ],
    )
    def k(feats_hbm, dense_hbm, t_hbm, out_hbm, idx_v, rows_v, dense_v, sem):
        wid = lax.axis_index("c") * NS + lax.axis_index("s")
        base = wid * W

        pltpu.sync_copy(dense_hbm.at[pl.ds(base, W), :], dense_v)
        pltpu.sync_copy(dense_v,
                        out_hbm.at[pl.ds(base, W), pl.ds(N_FIELDS * EMB, DENSE_DIM)])

        @pl.loop(0, N_FIELDS)
        def _field(f):
            pltpu.sync_copy(feats_hbm.at[f, pl.ds(base, W)], idx_v)

            @pl.loop(0, NCHUNK)
            def _chunk(j):
                pltpu.async_copy(
                    t_hbm.at[f].at[idx_v.at[pl.ds(j * CHUNK, CHUNK)]],
                    rows_v, sem).wait()
                pltpu.sync_copy(
                    rows_v,
                    out_hbm.at[pl.ds(base + j * CHUNK, CHUNK),
                               pl.ds(f * EMB, EMB)])

    return k(feats, dense, t)


def kernel(feat_0, feat_1, feat_2, feat_3, feat_4, feat_5, feat_6, feat_7,
           feat_8, feat_9, feat_10, feat_11, feat_12, feat_13, feat_14,
           feat_15, feat_16, feat_17, feat_18, feat_19, feat_20, feat_21,
           feat_22, feat_23, feat_24, feat_25, dense, tables):
    feats = jnp.stack([
        feat_0, feat_1, feat_2, feat_3, feat_4, feat_5, feat_6, feat_7,
        feat_8, feat_9, feat_10, feat_11, feat_12, feat_13, feat_14, feat_15,
        feat_16, feat_17, feat_18, feat_19, feat_20, feat_21, feat_22,
        feat_23, feat_24, feat_25,
    ]).astype(jnp.int32)
    t = _zero_pad_row(tables.astype(jnp.float32))
    return _sc_embed(feats, dense.astype(jnp.float32), t)


# speedup vs baseline: 4.3915x; 1.1909x over previous
import functools

import jax
import jax.numpy as jnp
from jax import lax
from jax.experimental import pallas as pl
from jax.experimental.pallas import tpu as pltpu
from jax.experimental.pallas import tpu_sc as plsc

N_FIELDS = 26
VOCAB_P1 = 1001
EMB = 128
BATCH = 16384
DENSE_DIM = 13
OUT_DIM = N_FIELDS * EMB + DENSE_DIM

NC, NS = 2, 16
NW = NC * NS
W = BATCH // NW
CHUNK = 64
NCHUNK = W // CHUNK


def _zero_pad_row(tables):
    def body(t_ref, o_ref):
        row = lax.broadcasted_iota(jnp.int32, (1, VOCAB_P1, EMB), 1)
        o_ref[...] = jnp.where(row == 0, 0.0, t_ref[...])

    return pl.pallas_call(
        body,
        grid=(N_FIELDS,),
        in_specs=[pl.BlockSpec((1, VOCAB_P1, EMB), lambda i: (i, 0, 0))],
        out_specs=pl.BlockSpec((1, VOCAB_P1, EMB), lambda i: (i, 0, 0)),
        out_shape=jax.ShapeDtypeStruct((N_FIELDS, VOCAB_P1, EMB), jnp.float32),
    )(tables)


NBUF = 4
TOTAL = N_FIELDS * NCHUNK


def _sc_embed(feats, dense, t):
    mesh = plsc.VectorSubcoreMesh(core_axis_name="c", subcore_axis_name="s")

    @functools.partial(
        pl.kernel,
        out_type=jax.ShapeDtypeStruct((BATCH, OUT_DIM), jnp.float32),
        mesh=mesh,
        scratch_types=[
            pltpu.VMEM((N_FIELDS, W), jnp.int32),
            pltpu.VMEM((NBUF, CHUNK, EMB), jnp.float32),
            pltpu.VMEM((W, DENSE_DIM), jnp.float32),
            pltpu.SemaphoreType.DMA,
            pltpu.SemaphoreType.DMA,
            pltpu.SemaphoreType.DMA,
        ],
    )
    def k(feats_hbm, dense_hbm, t_hbm, out_hbm, idx_v, bufs, dense_v,
          gsem, wsem, dsem):
        wid = lax.axis_index("c") * NS + lax.axis_index("s")
        base = wid * W

        dense_in = pltpu.async_copy(dense_hbm.at[pl.ds(base, W), :], dense_v,
                                    dsem)
        pltpu.sync_copy(feats_hbm.at[:, pl.ds(base, W)], idx_v)

        def gather_start(tt):
            f = tt // NCHUNK
            j = tt % NCHUNK
            pltpu.async_copy(
                t_hbm.at[f].at[idx_v.at[f, pl.ds(j * CHUNK, CHUNK)]],
                bufs.at[tt % NBUF], gsem)

        def write_start(tt):
            f = tt // NCHUNK
            j = tt % NCHUNK
            pltpu.async_copy(
                bufs.at[tt % NBUF],
                out_hbm.at[pl.ds(base + j * CHUNK, CHUNK),
                           pl.ds(f * EMB, EMB)], wsem)

        def gather_wait():
            pltpu.make_async_copy(t_hbm.at[0, pl.ds(0, CHUNK)], bufs.at[0],
                                  gsem).wait()

        def write_wait():
            pltpu.make_async_copy(bufs.at[0],
                                  out_hbm.at[pl.ds(base, CHUNK),
                                             pl.ds(0, EMB)], wsem).wait()

        for p in range(NBUF - 1):
            gather_start(p)

        @pl.loop(0, TOTAL)
        def _task(tt):
            gather_wait()
            write_start(tt)

            @pl.when(tt >= 1)
            def _():
                write_wait()

            @pl.when(tt + (NBUF - 1) < TOTAL)
            def _():
                gather_start(tt + (NBUF - 1))

        write_wait()
        dense_in.wait()
        pltpu.sync_copy(dense_v,
                        out_hbm.at[pl.ds(base, W), pl.ds(N_FIELDS * EMB, DENSE_DIM)])

    return k(feats, dense, t)


def kernel(feat_0, feat_1, feat_2, feat_3, feat_4, feat_5, feat_6, feat_7,
           feat_8, feat_9, feat_10, feat_11, feat_12, feat_13, feat_14,
           feat_15, feat_16, feat_17, feat_18, feat_19, feat_20, feat_21,
           feat_22, feat_23, feat_24, feat_25, dense, tables):
    feats = jnp.stack([
        feat_0, feat_1, feat_2, feat_3, feat_4, feat_5, feat_6, feat_7,
        feat_8, feat_9, feat_10, feat_11, feat_12, feat_13, feat_14, feat_15,
        feat_16, feat_17, feat_18, feat_19, feat_20, feat_21, feat_22,
        feat_23, feat_24, feat_25,
    ]).astype(jnp.int32)
    t = _zero_pad_row(tables.astype(jnp.float32))
    return _sc_embed(feats, dense.astype(jnp.float32), t)
